# Initial kernel scaffold; baseline (speedup 1.0000x reference)
import jax
import jax.numpy as jnp
from jax.experimental import pallas as pl


def kernel(x, table):
    raise NotImplementedError("write your pallas kernel here")


# speedup vs baseline: 3.4376x; 3.4376x over previous
import jax
import jax.numpy as jnp
from jax.experimental import pallas as pl


def _copy_body(t_ref, o_ref):
    o_ref[...] = t_ref[...]


def kernel(x, table):
    seq_len = x.shape[1]
    dim = table.shape[1]
    block_rows = 512
    return pl.pallas_call(
        _copy_body,
        grid=(seq_len // block_rows,),
        in_specs=[pl.BlockSpec((block_rows, dim), lambda i: (i, 0))],
        out_specs=pl.BlockSpec((block_rows, dim), lambda i: (i, 0)),
        out_shape=jax.ShapeDtypeStruct((seq_len, dim), table.dtype),
    )(table)
